# baseline (speedup 1.0000x reference)
import jax
import jax.numpy as jnp
from jax.experimental import pallas as pl
from jax.experimental.pallas import tpu as pltpu
from jax.experimental.pallas import tpu_sc as plsc

VOCAB = 100000
EMB = 64
CTX = 20
BATCH = 1024
HID = 256
NTOK = CTX * BATCH

GATHER_WINDOW = 128
V_TILE = 2048


NC = 2
NS = 16
NW = NC * NS
B_PER_W = NTOK // NW


EMBP = 128


def _sc_gather(table128, idx_flat):
    mesh = plsc.VectorSubcoreMesh(core_axis_name="c", subcore_axis_name="s")

    @pl.kernel(out_type=jax.ShapeDtypeStruct((NTOK, EMBP), table128.dtype),
               mesh=mesh,
               scratch_types=[
                   pltpu.VMEM((B_PER_W,), jnp.int32),
                   pltpu.VMEM((B_PER_W, EMBP), jnp.float32),
                   pltpu.SemaphoreType.DMA,
               ])
    def gather_kernel(table_hbm, idx_hbm, out_hbm, idx_v, rows_v, sem):
        wid = jax.lax.axis_index("s") * NC + jax.lax.axis_index("c")
        base = wid * B_PER_W
        pltpu.sync_copy(idx_hbm.at[pl.ds(base, B_PER_W)], idx_v)
        pltpu.async_copy(table_hbm.at[idx_v], rows_v, sem).wait()
        pltpu.sync_copy(rows_v, out_hbm.at[pl.ds(base, B_PER_W)])

    return gather_kernel(table128, idx_flat)


def _mm1_body(g_ref, w1_ref, b1_ref, h_ref):
    acc = b1_ref[...].astype(jnp.float32) * jnp.ones((BATCH, 1), jnp.float32)
    for k in range(CTX):
        xk = g_ref[pl.ds(1024 * k, 1024), pl.ds(0, EMB)]
        wk = w1_ref[:, pl.ds(EMB * k, EMB)]
        acc += jax.lax.dot_general(xk, wk, (((1,), (1,)), ((), ())),
                                   preferred_element_type=jnp.float32)
    h_ref[...] = jnp.tanh(acc)


def _mm2_body(h_ref, w2_ref, b2_ref, out_ref):
    acc = jax.lax.dot_general(w2_ref[...], h_ref[...], (((1,), (1,)), ((), ())),
                              preferred_element_type=jnp.float32)
    ones = jnp.ones((1, BATCH), jnp.float32)
    acc += jax.lax.dot_general(b2_ref[...], ones, (((0,), (0,)), ((), ())),
                               preferred_element_type=jnp.float32)
    out_ref[...] = acc


def kernel(inp, emb_table, W1, b1, W2, b2):
    idx_flat = inp.reshape(NTOK).astype(jnp.int32)
    idx_perm = idx_flat.reshape(BATCH, CTX).T.reshape(NTOK)
    table128 = jnp.concatenate([emb_table, emb_table], axis=1)
    gperm = _sc_gather(table128, idx_perm)

    h = pl.pallas_call(
        _mm1_body,
        out_shape=jax.ShapeDtypeStruct((BATCH, HID), jnp.float32),
    )(gperm, W1, b1.reshape(1, HID))

    n_tiles = pl.cdiv(VOCAB, V_TILE)
    out_t = pl.pallas_call(
        _mm2_body,
        grid=(n_tiles,),
        in_specs=[
            pl.BlockSpec((BATCH, HID), lambda i: (0, 0)),
            pl.BlockSpec((V_TILE, HID), lambda i: (i, 0)),
            pl.BlockSpec((1, V_TILE), lambda i: (0, i)),
        ],
        out_specs=pl.BlockSpec((V_TILE, BATCH), lambda i: (i, 0)),
        out_shape=jax.ShapeDtypeStruct((VOCAB, BATCH), jnp.float32),
        compiler_params=pltpu.CompilerParams(
            dimension_semantics=("parallel",)),
    )(h, W2, b2.reshape(1, VOCAB))
    return out_t.T


# speedup vs baseline: 1.0864x; 1.0864x over previous
import jax
import jax.numpy as jnp
from jax.experimental import pallas as pl
from jax.experimental.pallas import tpu as pltpu
from jax.experimental.pallas import tpu_sc as plsc

VOCAB = 100000
EMB = 64
CTX = 20
BATCH = 1024
HID = 256
NTOK = CTX * BATCH

GATHER_WINDOW = 128
V_TILE = 4096


NC = 2
NS = 16
NW = NC * NS
B_PER_W = NTOK // NW


EMBP = 128


def _sc_gather(table128, idx_flat):
    mesh = plsc.VectorSubcoreMesh(core_axis_name="c", subcore_axis_name="s")

    @pl.kernel(out_type=jax.ShapeDtypeStruct((NTOK, EMBP), table128.dtype),
               mesh=mesh,
               scratch_types=[
                   pltpu.VMEM((B_PER_W,), jnp.int32),
                   pltpu.VMEM((B_PER_W, EMBP), jnp.float32),
                   pltpu.SemaphoreType.DMA,
               ])
    def gather_kernel(table_hbm, idx_hbm, out_hbm, idx_v, rows_v, sem):
        wid = jax.lax.axis_index("s") * NC + jax.lax.axis_index("c")
        base = wid * B_PER_W
        pltpu.sync_copy(idx_hbm.at[pl.ds(base, B_PER_W)], idx_v)
        pltpu.async_copy(table_hbm.at[idx_v], rows_v, sem).wait()
        pltpu.sync_copy(rows_v, out_hbm.at[pl.ds(base, B_PER_W)])

    return gather_kernel(table128, idx_flat)


def _mm1_body(g_ref, w1_ref, b1_ref, h_ref):
    acc = b1_ref[...].astype(jnp.float32) * jnp.ones((BATCH, 1), jnp.float32)
    for k in range(CTX):
        xk = g_ref[pl.ds(1024 * k, 1024), pl.ds(0, EMB)]
        wk = w1_ref[:, pl.ds(EMB * k, EMB)]
        acc += jax.lax.dot_general(xk, wk, (((1,), (1,)), ((), ())),
                                   preferred_element_type=jnp.float32)
    h_ref[...] = jnp.tanh(acc)


def _mm2_body(h_ref, w2_ref, b2_ref, out_ref):
    acc = jax.lax.dot_general(w2_ref[...], h_ref[...], (((1,), (1,)), ((), ())),
                              preferred_element_type=jnp.float32)
    ones = jnp.ones((1, BATCH), jnp.float32)
    acc += jax.lax.dot_general(b2_ref[...], ones, (((0,), (0,)), ((), ())),
                               preferred_element_type=jnp.float32)
    out_ref[...] = acc


def kernel(inp, emb_table, W1, b1, W2, b2):
    idx_flat = inp.reshape(NTOK).astype(jnp.int32)
    idx_perm = idx_flat.reshape(BATCH, CTX).T.reshape(NTOK)
    table128 = jnp.pad(emb_table, ((0, 0), (0, EMBP - EMB)))
    gperm = _sc_gather(table128, idx_perm)

    h = pl.pallas_call(
        _mm1_body,
        out_shape=jax.ShapeDtypeStruct((BATCH, HID), jnp.float32),
    )(gperm, W1, b1.reshape(1, HID))

    n_tiles = pl.cdiv(VOCAB, V_TILE)
    out_t = pl.pallas_call(
        _mm2_body,
        grid=(n_tiles,),
        in_specs=[
            pl.BlockSpec((BATCH, HID), lambda i: (0, 0)),
            pl.BlockSpec((V_TILE, HID), lambda i: (i, 0)),
            pl.BlockSpec((1, V_TILE), lambda i: (0, i)),
        ],
        out_specs=pl.BlockSpec((V_TILE, BATCH), lambda i: (i, 0)),
        out_shape=jax.ShapeDtypeStruct((VOCAB, BATCH), jnp.float32),
        compiler_params=pltpu.CompilerParams(
            dimension_semantics=("parallel",)),
    )(h, W2, b2.reshape(1, VOCAB))
    return out_t.T


# speedup vs baseline: 1.0891x; 1.0024x over previous
import jax
import jax.numpy as jnp
from jax.experimental import pallas as pl
from jax.experimental.pallas import tpu as pltpu
from jax.experimental.pallas import tpu_sc as plsc

VOCAB = 100000
EMB = 64
CTX = 20
BATCH = 1024
HID = 256
NTOK = CTX * BATCH

GATHER_WINDOW = 128
V_TILE = 5120


NC = 2
NS = 16
NW = NC * NS
B_PER_W = NTOK // NW


EMBP = 128


def _sc_gather(table128, idx_flat):
    mesh = plsc.VectorSubcoreMesh(core_axis_name="c", subcore_axis_name="s")

    @pl.kernel(out_type=jax.ShapeDtypeStruct((NTOK, EMBP), table128.dtype),
               mesh=mesh,
               scratch_types=[
                   pltpu.VMEM((B_PER_W,), jnp.int32),
                   pltpu.VMEM((B_PER_W, EMBP), jnp.float32),
                   pltpu.SemaphoreType.DMA,
               ])
    def gather_kernel(table_hbm, idx_hbm, out_hbm, idx_v, rows_v, sem):
        wid = jax.lax.axis_index("s") * NC + jax.lax.axis_index("c")
        base = wid * B_PER_W
        pltpu.sync_copy(idx_hbm.at[pl.ds(base, B_PER_W)], idx_v)
        pltpu.async_copy(table_hbm.at[idx_v], rows_v, sem).wait()
        pltpu.sync_copy(rows_v, out_hbm.at[pl.ds(base, B_PER_W)])

    return gather_kernel(table128, idx_flat)


def _mm1_body(g_ref, w1_ref, b1_ref, h_ref):
    acc = b1_ref[...].astype(jnp.float32) * jnp.ones((BATCH, 1), jnp.float32)
    for k in range(CTX):
        xk = g_ref[pl.ds(1024 * k, 1024), pl.ds(0, EMB)]
        wk = w1_ref[:, pl.ds(EMB * k, EMB)]
        acc += jax.lax.dot_general(xk, wk, (((1,), (1,)), ((), ())),
                                   preferred_element_type=jnp.float32)
    h_ref[...] = jnp.tanh(acc)


def _mm2_body(h_ref, w2_ref, b2_ref, out_ref):
    acc = jax.lax.dot_general(w2_ref[...], h_ref[...], (((1,), (1,)), ((), ())),
                              preferred_element_type=jnp.float32)
    ones = jnp.ones((1, BATCH), jnp.float32)
    acc += jax.lax.dot_general(b2_ref[...], ones, (((0,), (0,)), ((), ())),
                               preferred_element_type=jnp.float32)
    out_ref[...] = acc


def kernel(inp, emb_table, W1, b1, W2, b2):
    idx_flat = inp.reshape(NTOK).astype(jnp.int32)
    idx_perm = idx_flat.reshape(BATCH, CTX).T.reshape(NTOK)
    table128 = jnp.pad(emb_table, ((0, 0), (0, EMBP - EMB)))
    gperm = _sc_gather(table128, idx_perm)

    h = pl.pallas_call(
        _mm1_body,
        out_shape=jax.ShapeDtypeStruct((BATCH, HID), jnp.float32),
    )(gperm, W1, b1.reshape(1, HID))

    n_tiles = pl.cdiv(VOCAB, V_TILE)
    out_t = pl.pallas_call(
        _mm2_body,
        grid=(n_tiles,),
        in_specs=[
            pl.BlockSpec((BATCH, HID), lambda i: (0, 0)),
            pl.BlockSpec((V_TILE, HID), lambda i: (i, 0)),
            pl.BlockSpec((1, V_TILE), lambda i: (0, i)),
        ],
        out_specs=pl.BlockSpec((V_TILE, BATCH), lambda i: (i, 0)),
        out_shape=jax.ShapeDtypeStruct((VOCAB, BATCH), jnp.float32),
        compiler_params=pltpu.CompilerParams(
            dimension_semantics=("parallel",)),
    )(h, W2, b2.reshape(1, VOCAB))
    return out_t.T
